# baseline (speedup 1.0000x reference)
import functools

import jax
import jax.numpy as jnp
import numpy as np
from jax.experimental import pallas as pl
from jax.experimental.pallas import tpu as pltpu

_N_BINS = 15
_BLOCK = 8192
_BOUNDS = np.array(
    [0x0, 0x3D888889, 0x3E088889, 0x3E4CCCCE, 0x3E888889, 0x3EAAAAAB,
     0x3ECCCCCE, 0x3EEEEEF0, 0x3F088889, 0x3F19999A, 0x3F2AAAAB,
     0x3F3BBBBC, 0x3F4CCCCE, 0x3F5DDDDF, 0x3F6EEEF0, 0x3F800000],
    dtype=np.uint32).view(np.float32)


def _pass1_kernel(n_total, x_ref, lab_ref, out_ref):
    i = pl.program_id(0)

    x = x_ref[...]
    lab = lab_ref[0]
    c_iota = jax.lax.broadcasted_iota(jnp.int32, x.shape, 0)
    conf = jnp.max(x, axis=0, keepdims=True)
    vlab = jnp.max(jnp.where(c_iota == lab, x, -jnp.inf), axis=0,
                   keepdims=True)
    acc = (vlab == conf).astype(jnp.float32)

    sub = _BLOCK // 8
    conf8 = conf.reshape(8, sub)
    acc8 = acc.reshape(8, sub)
    idx = (i * _BLOCK
           + jax.lax.broadcasted_iota(jnp.int32, (8, sub), 0) * sub
           + jax.lax.broadcasted_iota(jnp.int32, (8, sub), 1))
    conf8 = jnp.where(idx < jnp.int32(n_total), conf8, -1.0)

    cnts, scs, sas = [], [], []
    for j in range(_N_BINS + 1):
        m = (conf8 > _BOUNDS[j]).astype(jnp.float32)
        cnts.append(jnp.sum(m))
        scs.append(jnp.sum(conf8 * m))
        sas.append(jnp.sum(acc8 * m))
    out_ref[...] = jnp.stack(cnts + scs + sas).reshape(
        1, 1, 3 * (_N_BINS + 1))


def _pass2_kernel(n_total, part_ref, out_ref):
    tot = jnp.sum(part_ref[...], axis=(0, 1))
    nb16 = _N_BINS + 1
    tcnt = tot[:nb16]
    tsc = tot[nb16:2 * nb16]
    tsa = tot[2 * nb16:]
    cnt = tcnt[:_N_BINS] - tcnt[1:]
    sconf = tsc[:_N_BINS] - tsc[1:]
    sacc = tsa[:_N_BINS] - tsa[1:]
    safe = jnp.maximum(cnt, 1.0)
    contrib = jnp.abs(sconf / safe - sacc / safe) * (cnt / n_total)
    contrib = jnp.where(cnt > 0.0, contrib, 0.0)
    out_ref[...] = jnp.sum(contrib).reshape(1, 1)


def kernel(softmaxes, labels):
    n, c = softmaxes.shape
    nb = (n + _BLOCK - 1) // _BLOCK
    xt = softmaxes.T
    lab_pad = jnp.pad(labels.astype(jnp.int32), (0, nb * _BLOCK - n))
    lab3 = lab_pad.reshape(nb, 1, _BLOCK)
    width = 3 * (_N_BINS + 1)
    parts = pl.pallas_call(
        functools.partial(_pass1_kernel, float(n)),
        grid=(nb,),
        in_specs=[
            pl.BlockSpec((c, _BLOCK), lambda i: (0, i)),
            pl.BlockSpec((1, 1, _BLOCK), lambda i: (i, 0, 0)),
        ],
        out_specs=pl.BlockSpec((1, 1, width), lambda i: (i, 0, 0)),
        out_shape=jax.ShapeDtypeStruct((nb, 1, width), jnp.float32),
        compiler_params=pltpu.CompilerParams(
            dimension_semantics=("parallel",),
        ),
    )(xt, lab3)
    out = pl.pallas_call(
        functools.partial(_pass2_kernel, float(n)),
        in_specs=[pl.BlockSpec((nb, 1, width), lambda: (0, 0, 0))],
        out_specs=pl.BlockSpec((1, 1), lambda: (0, 0)),
        out_shape=jax.ShapeDtypeStruct((1, 1), softmaxes.dtype),
    )(parts)
    return out.reshape(1)


# speedup vs baseline: 1.2649x; 1.2649x over previous
import functools

import jax
import jax.numpy as jnp
import numpy as np
from jax.experimental import pallas as pl
from jax.experimental.pallas import tpu as pltpu

_N_BINS = 15
_BLOCK = 16384
_BOUNDS = np.array(
    [0x0, 0x3D888889, 0x3E088889, 0x3E4CCCCE, 0x3E888889, 0x3EAAAAAB,
     0x3ECCCCCE, 0x3EEEEEF0, 0x3F088889, 0x3F19999A, 0x3F2AAAAB,
     0x3F3BBBBC, 0x3F4CCCCE, 0x3F5DDDDF, 0x3F6EEEF0, 0x3F800000],
    dtype=np.uint32).view(np.float32)


def _ece_kernel(n_total, x_ref, lab_ref, out_ref, cnt_ref, sc_ref, sa_ref):
    i = pl.program_id(0)
    nb = pl.num_programs(0)

    @pl.when(i == 0)
    def _init():
        cnt_ref[...] = jnp.zeros_like(cnt_ref)
        sc_ref[...] = jnp.zeros_like(sc_ref)
        sa_ref[...] = jnp.zeros_like(sa_ref)
        out_ref[...] = jnp.zeros_like(out_ref)

    x = x_ref[...]
    lab = lab_ref[0]
    c_iota = jax.lax.broadcasted_iota(jnp.int32, x.shape, 0)
    conf = jnp.max(x, axis=0, keepdims=True)
    vlab = jnp.max(jnp.where(c_iota == lab, x, -jnp.inf), axis=0,
                   keepdims=True)
    acc = (vlab == conf).astype(jnp.float32)

    sub = _BLOCK // 8
    conf8 = conf.reshape(8, sub)
    acc8 = acc.reshape(8, sub)
    idx = (i * _BLOCK
           + jax.lax.broadcasted_iota(jnp.int32, (8, sub), 0) * sub
           + jax.lax.broadcasted_iota(jnp.int32, (8, sub), 1))
    conf8 = jnp.where(idx < jnp.int32(n_total), conf8, -1.0)

    for j in range(_N_BINS + 1):
        m = (conf8 > _BOUNDS[j]).astype(jnp.float32)
        cnt_ref[j] += m
        sc_ref[j] += conf8 * m
        sa_ref[j] += acc8 * m

    @pl.when(i == nb - 1)
    def _final():
        tcnt = jnp.sum(cnt_ref[...], axis=(1, 2))
        tsc = jnp.sum(sc_ref[...], axis=(1, 2))
        tsa = jnp.sum(sa_ref[...], axis=(1, 2))
        cnt = tcnt[:_N_BINS] - tcnt[1:]
        sconf = tsc[:_N_BINS] - tsc[1:]
        sacc = tsa[:_N_BINS] - tsa[1:]
        safe = jnp.maximum(cnt, 1.0)
        contrib = jnp.abs(sconf / safe - sacc / safe) * (cnt / n_total)
        contrib = jnp.where(cnt > 0.0, contrib, 0.0)
        out_ref[...] = jnp.sum(contrib).reshape(1, 1)


def kernel(softmaxes, labels):
    n, c = softmaxes.shape
    nb = (n + _BLOCK - 1) // _BLOCK
    xt = softmaxes.T
    lab_pad = jnp.pad(labels.astype(jnp.int32), (0, nb * _BLOCK - n))
    lab3 = lab_pad.reshape(nb, 1, _BLOCK)
    out = pl.pallas_call(
        functools.partial(_ece_kernel, float(n)),
        grid=(nb,),
        in_specs=[
            pl.BlockSpec((c, _BLOCK), lambda i: (0, i)),
            pl.BlockSpec((1, 1, _BLOCK), lambda i: (i, 0, 0)),
        ],
        out_specs=pl.BlockSpec((1, 1), lambda i: (0, 0)),
        out_shape=jax.ShapeDtypeStruct((1, 1), softmaxes.dtype),
        scratch_shapes=[
            pltpu.VMEM((_N_BINS + 1, 8, _BLOCK // 8), jnp.float32),
            pltpu.VMEM((_N_BINS + 1, 8, _BLOCK // 8), jnp.float32),
            pltpu.VMEM((_N_BINS + 1, 8, _BLOCK // 8), jnp.float32),
        ],
        compiler_params=pltpu.CompilerParams(
            dimension_semantics=("arbitrary",),
        ),
    )(xt, lab3)
    return out.reshape(1)


# speedup vs baseline: 1.3559x; 1.0719x over previous
import functools

import jax
import jax.numpy as jnp
import numpy as np
from jax.experimental import pallas as pl
from jax.experimental.pallas import tpu as pltpu

_N_BINS = 15
_BLOCK = 32768
_BOUNDS = np.array(
    [0x0, 0x3D888889, 0x3E088889, 0x3E4CCCCE, 0x3E888889, 0x3EAAAAAB,
     0x3ECCCCCE, 0x3EEEEEF0, 0x3F088889, 0x3F19999A, 0x3F2AAAAB,
     0x3F3BBBBC, 0x3F4CCCCE, 0x3F5DDDDF, 0x3F6EEEF0, 0x3F800000],
    dtype=np.uint32).view(np.float32)


def _ece_kernel(n_total, x_ref, lab_ref, out_ref, cnt_ref, sc_ref, sa_ref):
    i = pl.program_id(0)
    nb = pl.num_programs(0)

    @pl.when(i == 0)
    def _init():
        cnt_ref[...] = jnp.zeros_like(cnt_ref)
        sc_ref[...] = jnp.zeros_like(sc_ref)
        sa_ref[...] = jnp.zeros_like(sa_ref)
        out_ref[...] = jnp.zeros_like(out_ref)

    x = x_ref[...]
    lab = lab_ref[0]
    c_iota = jax.lax.broadcasted_iota(jnp.int32, x.shape, 0)
    conf = jnp.max(x, axis=0, keepdims=True)
    vlab = jnp.max(jnp.where(c_iota == lab, x, -jnp.inf), axis=0,
                   keepdims=True)
    acc = (vlab == conf).astype(jnp.float32)

    sub = _BLOCK // 8
    conf8 = conf.reshape(8, sub)
    acc8 = acc.reshape(8, sub)
    idx = (i * _BLOCK
           + jax.lax.broadcasted_iota(jnp.int32, (8, sub), 0) * sub
           + jax.lax.broadcasted_iota(jnp.int32, (8, sub), 1))
    conf8 = jnp.where(idx < jnp.int32(n_total), conf8, -1.0)

    for j in range(_N_BINS + 1):
        m = (conf8 > _BOUNDS[j]).astype(jnp.float32)
        cnt_ref[j] += m
        sc_ref[j] += conf8 * m
        sa_ref[j] += acc8 * m

    @pl.when(i == nb - 1)
    def _final():
        tcnt = jnp.sum(cnt_ref[...], axis=(1, 2))
        tsc = jnp.sum(sc_ref[...], axis=(1, 2))
        tsa = jnp.sum(sa_ref[...], axis=(1, 2))
        cnt = tcnt[:_N_BINS] - tcnt[1:]
        sconf = tsc[:_N_BINS] - tsc[1:]
        sacc = tsa[:_N_BINS] - tsa[1:]
        safe = jnp.maximum(cnt, 1.0)
        contrib = jnp.abs(sconf / safe - sacc / safe) * (cnt / n_total)
        contrib = jnp.where(cnt > 0.0, contrib, 0.0)
        out_ref[...] = jnp.sum(contrib).reshape(1, 1)


def kernel(softmaxes, labels):
    n, c = softmaxes.shape
    nb = (n + _BLOCK - 1) // _BLOCK
    xt = softmaxes.T
    lab_pad = jnp.pad(labels.astype(jnp.int32), (0, nb * _BLOCK - n))
    lab3 = lab_pad.reshape(nb, 1, _BLOCK)
    out = pl.pallas_call(
        functools.partial(_ece_kernel, float(n)),
        grid=(nb,),
        in_specs=[
            pl.BlockSpec((c, _BLOCK), lambda i: (0, i)),
            pl.BlockSpec((1, 1, _BLOCK), lambda i: (i, 0, 0)),
        ],
        out_specs=pl.BlockSpec((1, 1), lambda i: (0, 0)),
        out_shape=jax.ShapeDtypeStruct((1, 1), softmaxes.dtype),
        scratch_shapes=[
            pltpu.VMEM((_N_BINS + 1, 8, _BLOCK // 8), jnp.float32),
            pltpu.VMEM((_N_BINS + 1, 8, _BLOCK // 8), jnp.float32),
            pltpu.VMEM((_N_BINS + 1, 8, _BLOCK // 8), jnp.float32),
        ],
        compiler_params=pltpu.CompilerParams(
            dimension_semantics=("arbitrary",),
        ),
    )(xt, lab3)
    return out.reshape(1)
